# baseline (speedup 1.0000x reference)
import jax
import jax.numpy as jnp
from jax import lax
from jax.experimental import pallas as pl
from jax.experimental.pallas import tpu as pltpu

A = 250
D = 128
H = 128
KP1 = 17.0
BPP = 4

_DN_STD = (((1,), (0,)), ((), ()))


def _mm(a, b, dn, precision):
    return lax.dot_general(a, b, dn, precision=precision,
                           preferred_element_type=jnp.float32)


def _dist2(o):
    posT = jnp.transpose(o[:, 0:8])
    px_r = posT[0:1, :]
    py_r = posT[1:2, :]
    dx = o[:, 0:1] - px_r
    dy = o[:, 1:2] - py_r
    return dx * dx + dy * dy


def _graph_tail(d, thr, adj_ref):
    iota_ci = lax.broadcasted_iota(jnp.int32, (A, A), 1)
    iota_ri = lax.broadcasted_iota(jnp.int32, (A, A), 0)
    less = d < thr
    tie = d == thr
    zero = d == 0.0
    cnt_less = jnp.sum(jnp.where(less, 1.0, 0.0), axis=0, keepdims=True)
    tcount = jnp.sum(jnp.where(tie, 1.0, 0.0), axis=0, keepdims=True)
    zcount = jnp.sum(jnp.where(zero, 1.0, 0.0), axis=0, keepdims=True)
    need = KP1 - cnt_less
    offdiag = iota_ri != iota_ci
    sel_fast = jnp.logical_and(jnp.logical_or(less, tie), offdiag)
    adj_ref[...] = jnp.where(sel_fast, 1.0, 0.0)
    bad = (jnp.max(jnp.abs(tcount - need)) + jnp.max(jnp.abs(zcount - 1.0))
           ) > 0.5

    @pl.when(bad)
    def _exact_tail():
        iota_r = iota_ri.astype(jnp.float32)
        tie_f = jnp.where(tie, 1.0, 0.0)
        tri_low = jnp.where(iota_ci < iota_ri, 1.0, 0.0)
        tie_rank = _mm(tri_low, tie_f, _DN_STD, lax.Precision.DEFAULT)
        sel = jnp.logical_or(less, jnp.logical_and(tie, tie_rank < need))
        first_idx = jnp.min(jnp.where(zero, iota_r, jnp.float32(A)),
                            axis=0, keepdims=True)
        sel = jnp.logical_and(sel, iota_r != first_idx)
        adj_ref[...] = jnp.where(sel, 1.0, 0.0)

    adjT = adj_ref[...]
    eye = jnp.where(iota_ri == iota_ci, 1.0, 0.0)
    deg = jnp.sum(adjT, axis=1, keepdims=True) + 1.0
    dis = 1.0 / jnp.sqrt(deg)
    adjn = (adjT + eye).astype(jnp.bfloat16)
    return adjn, dis


def _thresholds(ds, thr_ref):
    inf = jnp.float32(jnp.inf)

    def fast_body(_, ms):
        return tuple(
            jnp.min(jnp.where(d > m, d, inf), axis=0, keepdims=True)
            for d, m in zip(ds, ms))

    m0 = tuple(jnp.zeros((1, A), jnp.float32) for _ in ds)
    m17s = lax.fori_loop(0, 16, fast_body, m0)

    for u, (d, m17) in enumerate(zip(ds, m17s)):
        thr_ref[u] = m17
        cnt_less = jnp.sum(jnp.where(d < m17, 1.0, 0.0), axis=0, keepdims=True)
        bad = jnp.max(cnt_less) > 16.5

        @pl.when(bad)
        def _slow_path(d=d, u=u):
            def body(_, carry):
                d_m, cnt, thr = carry
                m = jnp.min(d_m, axis=0, keepdims=True)
                eqm = d_m == m
                ceq = jnp.sum(jnp.where(eqm, 1.0, 0.0), axis=0,
                              keepdims=True)
                thr = jnp.where(cnt < KP1, m, thr)
                cnt = cnt + ceq
                d_m = jnp.where(eqm, inf, d_m)
                return d_m, cnt, thr

            zeros_r = jnp.zeros((1, A), jnp.float32)
            _, _, thr_s = lax.fori_loop(0, 17, body, (d, zeros_r, zeros_r))
            thr_ref[u] = thr_s

    return [thr_ref[u] for u in range(len(ds))]


def _conv(adjn, dis, y):
    dflt = lax.Precision.DEFAULT
    y = y * dis
    y1 = y.astype(jnp.bfloat16)
    y2 = (y - y1.astype(jnp.float32)).astype(jnp.bfloat16)
    z = _mm(adjn, y1, _DN_STD, dflt) + _mm(adjn, y2, _DN_STD, dflt)
    return z * dis


def _gnn_body(obs_ref, w1_ref, b1_ref, w2_ref, b2_ref, wo_ref, bo_ref, out_ref,
              thr_ref, adj_ref):
    dflt = lax.Precision.DEFAULT
    w1 = w1_ref[...]
    b1 = b1_ref[...]
    w2 = w2_ref[...]
    b2 = b2_ref[...]
    wo = wo_ref[...]
    bo = bo_ref[...]
    ds = [_dist2(obs_ref[u]) for u in range(BPP)]
    thrs = _thresholds(ds, thr_ref)
    graphs = [_graph_tail(d, thr, adj_ref.at[u])
              for u, (d, thr) in enumerate(zip(ds, thrs))]
    for u in range(BPP):
        o = obs_ref[u]
        adjn, dis = graphs[u]
        xw = _mm(o, w1, _DN_STD, dflt)
        h = jnp.tanh(_conv(adjn, dis, xw) + b1)
        xw2 = _mm(h, w2, _DN_STD, dflt)
        h2 = jnp.tanh(_conv(adjn, dis, xw2) + b2)
        out_ref[u] = _mm(h2, wo, _DN_STD, dflt) + bo


@jax.jit
def kernel(agent_observations, W1, b1, W2, b2, Wo, bo):
    obs = agent_observations.astype(jnp.float32)
    B = obs.shape[0]
    out = pl.pallas_call(
        _gnn_body,
        grid=(B // BPP,),
        in_specs=[
            pl.BlockSpec((BPP, A, D), lambda b: (b, 0, 0)),
            pl.BlockSpec((D, H), lambda b: (0, 0)),
            pl.BlockSpec((1, H), lambda b: (0, 0)),
            pl.BlockSpec((H, H), lambda b: (0, 0)),
            pl.BlockSpec((1, H), lambda b: (0, 0)),
            pl.BlockSpec((H, 1), lambda b: (0, 0)),
            pl.BlockSpec((1, 1), lambda b: (0, 0)),
        ],
        out_specs=pl.BlockSpec((BPP, A, 1), lambda b: (b, 0, 0)),
        out_shape=jax.ShapeDtypeStruct((B, A, 1), jnp.float32),
        scratch_shapes=[pltpu.VMEM((BPP, 1, A), jnp.float32),
                        pltpu.VMEM((BPP, A, A), jnp.float32)],
    )(obs, W1, b1.reshape(1, H), W2, b2.reshape(1, H), Wo, bo.reshape(1, 1))
    return out


# speedup vs baseline: 1.0776x; 1.0776x over previous
import jax
import jax.numpy as jnp
from jax import lax
from jax.experimental import pallas as pl
from jax.experimental.pallas import tpu as pltpu

A = 250
D = 128
H = 128
KP1 = 17.0
BPP = 4

_DN_STD = (((1,), (0,)), ((), ()))


def _mm(a, b, dn, precision):
    return lax.dot_general(a, b, dn, precision=precision,
                           preferred_element_type=jnp.float32)


def _dist2(o):
    posT = jnp.transpose(o[:, 0:8])
    px_r = posT[0:1, :]
    py_r = posT[1:2, :]
    dx = o[:, 0:1] - px_r
    dy = o[:, 1:2] - py_r
    return dx * dx + dy * dy


def _graph_tail(d, thr):
    iota_ci = lax.broadcasted_iota(jnp.int32, (A, A), 1)
    iota_ri = lax.broadcasted_iota(jnp.int32, (A, A), 0)
    iota_r = iota_ri.astype(jnp.float32)
    less = d < thr
    cnt_less = jnp.sum(jnp.where(less, 1.0, 0.0), axis=0, keepdims=True)
    tie = d == thr
    tie_f = jnp.where(tie, 1.0, 0.0)
    tri_low = jnp.where(iota_ci < iota_ri, 1.0, 0.0)
    tie_rank = _mm(tri_low, tie_f, _DN_STD, lax.Precision.DEFAULT)
    need = KP1 - cnt_less
    sel = jnp.logical_or(less, jnp.logical_and(tie, tie_rank < need))
    first_idx = jnp.min(jnp.where(d == 0.0, iota_r, jnp.float32(A)),
                        axis=0, keepdims=True)
    sel = jnp.logical_and(sel, iota_r != first_idx)
    adjT = jnp.where(sel, 1.0, 0.0)

    eye = jnp.where(iota_ri == iota_ci, 1.0, 0.0)
    deg = jnp.sum(adjT, axis=1, keepdims=True) + 1.0
    dis = 1.0 / jnp.sqrt(deg)
    adjn = (adjT + eye).astype(jnp.bfloat16)
    return adjn, dis


def _thresholds(ds, thr_ref):
    inf = jnp.float32(jnp.inf)

    def fast_body(_, ms):
        return tuple(
            jnp.min(jnp.where(d > m, d, inf), axis=0, keepdims=True)
            for d, m in zip(ds, ms))

    m0 = tuple(jnp.zeros((1, A), jnp.float32) for _ in ds)
    m17s = lax.fori_loop(0, 16, fast_body, m0)

    for u, (d, m17) in enumerate(zip(ds, m17s)):
        thr_ref[u] = m17
        cnt_less = jnp.sum(jnp.where(d < m17, 1.0, 0.0), axis=0, keepdims=True)
        bad = jnp.max(cnt_less) > 16.5

        @pl.when(bad)
        def _slow_path(d=d, u=u):
            def body(_, carry):
                d_m, cnt, thr = carry
                m = jnp.min(d_m, axis=0, keepdims=True)
                eqm = d_m == m
                ceq = jnp.sum(jnp.where(eqm, 1.0, 0.0), axis=0,
                              keepdims=True)
                thr = jnp.where(cnt < KP1, m, thr)
                cnt = cnt + ceq
                d_m = jnp.where(eqm, inf, d_m)
                return d_m, cnt, thr

            zeros_r = jnp.zeros((1, A), jnp.float32)
            _, _, thr_s = lax.fori_loop(0, 17, body, (d, zeros_r, zeros_r))
            thr_ref[u] = thr_s

    return [thr_ref[u] for u in range(len(ds))]


def _conv(adjn, dis, y):
    dflt = lax.Precision.DEFAULT
    y = y * dis
    y1 = y.astype(jnp.bfloat16)
    y2 = (y - y1.astype(jnp.float32)).astype(jnp.bfloat16)
    z = _mm(adjn, y1, _DN_STD, dflt) + _mm(adjn, y2, _DN_STD, dflt)
    return z * dis


def _gnn_body(obs_ref, w1_ref, b1_ref, w2_ref, b2_ref, wo_ref, bo_ref, out_ref,
              thr_ref):
    dflt = lax.Precision.DEFAULT
    w1 = w1_ref[...]
    b1 = b1_ref[...]
    w2 = w2_ref[...]
    b2 = b2_ref[...]
    wo = wo_ref[...]
    bo = bo_ref[...]
    ds = [_dist2(obs_ref[u]) for u in range(BPP)]
    thrs = _thresholds(ds, thr_ref)
    graphs = [_graph_tail(d, thr) for d, thr in zip(ds, thrs)]
    for u in range(BPP):
        o = obs_ref[u]
        adjn, dis = graphs[u]
        xw = _mm(o, w1, _DN_STD, dflt)
        h = jnp.tanh(_conv(adjn, dis, xw) + b1)
        xw2 = _mm(h, w2, _DN_STD, dflt)
        h2 = jnp.tanh(_conv(adjn, dis, xw2) + b2)
        out_ref[u] = _mm(h2, wo, _DN_STD, dflt) + bo


@jax.jit
def kernel(agent_observations, W1, b1, W2, b2, Wo, bo):
    obs = agent_observations.astype(jnp.float32)
    B = obs.shape[0]
    out = pl.pallas_call(
        _gnn_body,
        grid=(B // BPP,),
        in_specs=[
            pl.BlockSpec((BPP, A, D), lambda b: (b, 0, 0)),
            pl.BlockSpec((D, H), lambda b: (0, 0)),
            pl.BlockSpec((1, H), lambda b: (0, 0)),
            pl.BlockSpec((H, H), lambda b: (0, 0)),
            pl.BlockSpec((1, H), lambda b: (0, 0)),
            pl.BlockSpec((H, 1), lambda b: (0, 0)),
            pl.BlockSpec((1, 1), lambda b: (0, 0)),
        ],
        out_specs=pl.BlockSpec((BPP, A, 1), lambda b: (b, 0, 0)),
        out_shape=jax.ShapeDtypeStruct((B, A, 1), jnp.float32),
        scratch_shapes=[pltpu.VMEM((BPP, 1, A), jnp.float32)],
    )(obs, W1, b1.reshape(1, H), W2, b2.reshape(1, H), Wo, bo.reshape(1, 1))
    return out
